# baseline (speedup 1.0000x reference)
import functools

import numpy as np
import jax
import jax.numpy as jnp
from jax.experimental import pallas as pl
from jax.experimental.pallas import tpu as pltpu

_SUB = 8
_W = 512
_CH = _SUB * _W

_INT_MAX = np.int32(0x7FFFFFFF)


def _threefry_score(p, v, base_u, k0, k1, ks2, width):
    i = base_u + v.astype(jnp.uint32)
    ks = (k0, k1, ks2)
    x0 = jnp.broadcast_to(k0, (_SUB, width))
    x1 = i + k1
    rotations = ((13, 15, 26, 6), (17, 29, 16, 24))
    for g in range(5):
        for r in rotations[g % 2]:
            x0 = x0 + x1
            x1 = (x1 << jnp.uint32(r)) | (x1 >> jnp.uint32(32 - r))
            x1 = x1 ^ x0
        x0 = x0 + ks[(g + 1) % 3]
        x1 = x1 + (ks[(g + 2) % 3] + jnp.uint32(g + 1))
    bits = x0 ^ x1
    f = jax.lax.bitcast_convert_type(
        (bits >> jnp.uint32(9)) | jnp.uint32(0x3F800000), jnp.float32
    ) - jnp.float32(1.0)
    tiny = jnp.float32(np.finfo(np.float32).tiny)
    t = -jnp.log(jnp.maximum(f, tiny))
    return p / t


def _sample_kernel(sidx_ref, order_ref, key_ref, p_ref, a_ref, lp_ref, sp_ref, *, vocab):
    b = pl.program_id(0)
    ob = order_ref[b]
    row = sidx_ref[b]
    s = row - (row // _SUB) * _SUB

    k0 = key_ref[0]
    k1 = key_ref[1]
    ks2 = k0 ^ k1 ^ jnp.uint32(0x1BD11BDA)
    base_u = (ob * jnp.int32(vocab)).astype(jnp.uint32)

    nfull = vocab // _CH
    tailn = vocab - nfull * _CH
    tail_w = tailn // _SUB

    rowi = jax.lax.broadcasted_iota(jnp.int32, (_SUB, _W), 0)
    lane = jax.lax.broadcasted_iota(jnp.int32, (_SUB, _W), 1)
    v0 = rowi * jnp.int32(_W) + lane

    m_acc = jnp.full((_SUB, _W), -jnp.inf, jnp.float32)
    a_acc = jnp.zeros((_SUB, _W), jnp.int32)
    ps_acc = jnp.zeros((_SUB, _W), jnp.float32)
    s_acc = jnp.zeros((_SUB, _W), jnp.float32)

    def extract(c0, width):
        pieces = [
            p_ref[0, pl.ds(s, 1), c0 + r * width:c0 + (r + 1) * width]
            for r in range(_SUB)
        ]
        return jnp.concatenate(pieces, axis=0)

    for kc in range(nfull):
        v = v0 + jnp.int32(kc * _CH)
        p = extract(kc * _CH, _W)
        sc = _threefry_score(p, v, base_u, k0, k1, ks2, _W)
        upd = sc > m_acc
        m_acc = jnp.where(upd, sc, m_acc)
        a_acc = jnp.where(upd, v, a_acc)
        ps_acc = jnp.where(upd, p, ps_acc)
        s_acc = s_acc + p

    m1 = jnp.max(m_acc)
    win = m_acc == m1
    a1 = jnp.min(jnp.where(win, a_acc, _INT_MAX))
    sel1 = jnp.sum(jnp.where(win & (a_acc == a1), ps_acc, jnp.float32(0.0)))
    total = jnp.sum(s_acc)

    if tailn:
        rowt = jax.lax.broadcasted_iota(jnp.int32, (_SUB, tail_w), 0)
        lanet = jax.lax.broadcasted_iota(jnp.int32, (_SUB, tail_w), 1)
        vt = rowt * jnp.int32(tail_w) + lanet + jnp.int32(nfull * _CH)
        pt = extract(nfull * _CH, tail_w)
        st = _threefry_score(pt, vt, base_u, k0, k1, ks2, tail_w)
        m2 = jnp.max(st)
        a2 = jnp.min(jnp.where(st == m2, vt, _INT_MAX))
        sel2 = jnp.sum(jnp.where(vt == a2, pt, jnp.float32(0.0)))
        total = total + jnp.sum(pt)
        better2 = (m2 > m1) | ((m2 == m1) & (a2 < a1))
        a1 = jnp.where(better2, a2, a1)
        sel1 = jnp.where(better2, sel2, sel1)

    a_ref[0, 0, 0] = a1
    lp_ref[0, 0, 0] = jnp.log(sel1) - jnp.log(total)
    sp_ref[0, 0, 0] = sel1


def kernel(probs_table, indices):
    num_states, vocab = probs_table.shape
    batch = indices.shape[0]
    assert num_states % _SUB == 0 and vocab % _SUB == 0
    table_g = probs_table.reshape(num_states // _SUB, _SUB, vocab)

    key_data = jax.random.key_data(
        jax.random.fold_in(jax.random.key(0), 123)
    ).astype(jnp.uint32)

    order = jnp.argsort(indices).astype(jnp.int32)
    sidx = jnp.take(indices, order)

    grid_spec = pltpu.PrefetchScalarGridSpec(
        num_scalar_prefetch=3,
        grid=(batch,),
        in_specs=[
            pl.BlockSpec(
                (1, _SUB, vocab),
                lambda b, sidx, order, key: (sidx[b] // _SUB, 0, 0),
            ),
        ],
        out_specs=[
            pl.BlockSpec((1, 1, 1), lambda b, sidx, order, key: (order[b], 0, 0), memory_space=pltpu.SMEM),
            pl.BlockSpec((1, 1, 1), lambda b, sidx, order, key: (order[b], 0, 0), memory_space=pltpu.SMEM),
            pl.BlockSpec((1, 1, 1), lambda b, sidx, order, key: (order[b], 0, 0), memory_space=pltpu.SMEM),
        ],
    )
    actions, log_probs, selected = pl.pallas_call(
        functools.partial(_sample_kernel, vocab=vocab),
        grid_spec=grid_spec,
        out_shape=[
            jax.ShapeDtypeStruct((batch, 1, 1), jnp.int32),
            jax.ShapeDtypeStruct((batch, 1, 1), jnp.float32),
            jax.ShapeDtypeStruct((batch, 1, 1), jnp.float32),
        ],
    )(sidx, order, key_data, table_g)
    return actions[:, 0, 0], log_probs[:, 0, 0], selected[:, 0, 0]


# speedup vs baseline: 1.0034x; 1.0034x over previous
import functools

import numpy as np
import jax
import jax.numpy as jnp
from jax.experimental import pallas as pl
from jax.experimental.pallas import tpu as pltpu

_SUB = 8
_W = 512
_CH = _SUB * _W

_INT_MAX = np.int32(0x7FFFFFFF)


def _threefry_score(p, v, base_u, k0, k1, ks2, width):
    i = base_u + v.astype(jnp.uint32)
    ks = (k0, k1, ks2)
    x0 = jnp.broadcast_to(k0, (_SUB, width))
    x1 = i + k1
    rotations = ((13, 15, 26, 6), (17, 29, 16, 24))
    for g in range(5):
        for r in rotations[g % 2]:
            x0 = x0 + x1
            x1 = (x1 << jnp.uint32(r)) | (x1 >> jnp.uint32(32 - r))
            x1 = x1 ^ x0
        x0 = x0 + ks[(g + 1) % 3]
        x1 = x1 + (ks[(g + 2) % 3] + jnp.uint32(g + 1))
    bits = x0 ^ x1
    f = jax.lax.bitcast_convert_type(
        (bits >> jnp.uint32(9)) | jnp.uint32(0x3F800000), jnp.float32
    ) - jnp.float32(1.0)
    tiny = jnp.float32(np.finfo(np.float32).tiny)
    t = -jnp.log(jnp.maximum(f, tiny))
    return p / t


def _sample_kernel(sidx_ref, order_ref, fetch_ref, slot_ref, key_ref,
                   tab_ref, a_ref, lp_ref, sp_ref, buf, sems, *, vocab):
    b = pl.program_id(0)
    nb = pl.num_programs(0)

    def group_copy(step):
        g = sidx_ref[step] // _SUB
        sl = slot_ref[step]
        return pltpu.make_async_copy(tab_ref.at[g], buf.at[sl], sems.at[sl])

    @pl.when(b == 0)
    def _():
        group_copy(0).start()

    @pl.when((b + 1 < nb) & (fetch_ref[b + 1] == 1))
    def _():
        group_copy(b + 1).start()

    @pl.when(fetch_ref[b] == 1)
    def _():
        group_copy(b).wait()

    slot = slot_ref[b]
    ob = order_ref[b]
    row = sidx_ref[b]
    s = row - (row // _SUB) * _SUB

    k0 = key_ref[0]
    k1 = key_ref[1]
    ks2 = k0 ^ k1 ^ jnp.uint32(0x1BD11BDA)
    base_u = (ob * jnp.int32(vocab)).astype(jnp.uint32)

    nfull = vocab // _CH
    tailn = vocab - nfull * _CH
    tail_w = tailn // _SUB

    rowi = jax.lax.broadcasted_iota(jnp.int32, (_SUB, _W), 0)
    lane = jax.lax.broadcasted_iota(jnp.int32, (_SUB, _W), 1)
    v0 = rowi * jnp.int32(_W) + lane

    m_acc = jnp.full((_SUB, _W), -jnp.inf, jnp.float32)
    a_acc = jnp.zeros((_SUB, _W), jnp.int32)
    ps_acc = jnp.zeros((_SUB, _W), jnp.float32)
    s_acc = jnp.zeros((_SUB, _W), jnp.float32)

    def extract(c0, width):
        pieces = [
            buf[slot, pl.ds(s, 1), c0 + r * width:c0 + (r + 1) * width]
            for r in range(_SUB)
        ]
        return jnp.concatenate(pieces, axis=0)

    for kc in range(nfull):
        v = v0 + jnp.int32(kc * _CH)
        p = extract(kc * _CH, _W)
        sc = _threefry_score(p, v, base_u, k0, k1, ks2, _W)
        upd = sc > m_acc
        m_acc = jnp.where(upd, sc, m_acc)
        a_acc = jnp.where(upd, v, a_acc)
        ps_acc = jnp.where(upd, p, ps_acc)
        s_acc = s_acc + p

    m1 = jnp.max(m_acc)
    win = m_acc == m1
    a1 = jnp.min(jnp.where(win, a_acc, _INT_MAX))
    sel1 = jnp.sum(jnp.where(win & (a_acc == a1), ps_acc, jnp.float32(0.0)))
    total = jnp.sum(s_acc)

    if tailn:
        rowt = jax.lax.broadcasted_iota(jnp.int32, (_SUB, tail_w), 0)
        lanet = jax.lax.broadcasted_iota(jnp.int32, (_SUB, tail_w), 1)
        vt = rowt * jnp.int32(tail_w) + lanet + jnp.int32(nfull * _CH)
        pt = extract(nfull * _CH, tail_w)
        st = _threefry_score(pt, vt, base_u, k0, k1, ks2, tail_w)
        m2 = jnp.max(st)
        a2 = jnp.min(jnp.where(st == m2, vt, _INT_MAX))
        sel2 = jnp.sum(jnp.where(vt == a2, pt, jnp.float32(0.0)))
        total = total + jnp.sum(pt)
        better2 = (m2 > m1) | ((m2 == m1) & (a2 < a1))
        a1 = jnp.where(better2, a2, a1)
        sel1 = jnp.where(better2, sel2, sel1)

    a_ref[0, 0, 0] = a1
    lp_ref[0, 0, 0] = jnp.log(sel1) - jnp.log(total)
    sp_ref[0, 0, 0] = sel1


def kernel(probs_table, indices):
    num_states, vocab = probs_table.shape
    batch = indices.shape[0]
    assert num_states % _SUB == 0 and vocab % _SUB == 0
    table_g = probs_table.reshape(num_states // _SUB, _SUB, vocab)

    key_data = jax.random.key_data(
        jax.random.fold_in(jax.random.key(0), 123)
    ).astype(jnp.uint32)

    order = jnp.argsort(indices).astype(jnp.int32)
    sidx = jnp.take(indices, order)
    grp = sidx // _SUB
    change = jnp.concatenate(
        [jnp.ones((1,), jnp.int32), (grp[1:] != grp[:-1]).astype(jnp.int32)]
    )
    fetch = jnp.concatenate([change, jnp.zeros((1,), jnp.int32)])
    slot = (jnp.cumsum(change) - 1) % 2

    grid_spec = pltpu.PrefetchScalarGridSpec(
        num_scalar_prefetch=5,
        grid=(batch,),
        in_specs=[pl.BlockSpec(memory_space=pl.ANY)],
        out_specs=[
            pl.BlockSpec((1, 1, 1), lambda b, sidx, order, fetch, slot, key: (order[b], 0, 0), memory_space=pltpu.SMEM),
            pl.BlockSpec((1, 1, 1), lambda b, sidx, order, fetch, slot, key: (order[b], 0, 0), memory_space=pltpu.SMEM),
            pl.BlockSpec((1, 1, 1), lambda b, sidx, order, fetch, slot, key: (order[b], 0, 0), memory_space=pltpu.SMEM),
        ],
        scratch_shapes=[
            pltpu.VMEM((2, _SUB, vocab), jnp.float32),
            pltpu.SemaphoreType.DMA((2,)),
        ],
    )
    actions, log_probs, selected = pl.pallas_call(
        functools.partial(_sample_kernel, vocab=vocab),
        grid_spec=grid_spec,
        out_shape=[
            jax.ShapeDtypeStruct((batch, 1, 1), jnp.int32),
            jax.ShapeDtypeStruct((batch, 1, 1), jnp.float32),
            jax.ShapeDtypeStruct((batch, 1, 1), jnp.float32),
        ],
    )(sidx, order, fetch.astype(jnp.int32), slot.astype(jnp.int32), key_data, table_g)
    return actions[:, 0, 0], log_probs[:, 0, 0], selected[:, 0, 0]
